# baseline (speedup 1.0000x reference)
import jax
import jax.numpy as jnp
from jax import lax
from jax.experimental import pallas as pl
from jax.experimental.pallas import tpu as pltpu
from jax.experimental.pallas import tpu_sc as plsc

N = 10000
D = 128
E = 320000

NC = 2
NS = 16
DH = D // NC

CHUNK = 128
NCHUNK = 158
EDGES_PER_TILE = NCHUNK * CHUNK
EPAD = EDGES_PER_TILE * NS
NBUF = 4
LOOK = 2
ROWS_PER_TILE = -(-N // (NS * CHUNK)) * CHUNK
NPAD = ROWS_PER_TILE * NS


def _sc_body(edges_hbm, x_hbm, out_hbm, edges_v, rows_v, acc, sem_g, sem_s):
    c = lax.axis_index("c")
    s = lax.axis_index("s")

    @pl.loop(0, CHUNK)
    def _zero_rows(i):
        for j in range(DH // 16):
            rows_v[0, i, pl.ds(j * 16, 16)] = jnp.zeros((16,), jnp.float32)

    for j in range(ROWS_PER_TILE // CHUNK):
        pltpu.sync_copy(
            rows_v.at[0], acc.at[pl.ds(s * ROWS_PER_TILE + j * CHUNK, CHUNK)]
        )
    plsc.subcore_barrier()

    def start_gather(g, b):
        pltpu.sync_copy(edges_hbm.at[s, g], edges_v.at[b])
        pltpu.async_copy(
            x_hbm.at[c].at[edges_v.at[b, 0]], rows_v.at[b], sem_g.at[b]
        )

    def wait_gather(b):
        pltpu.make_async_copy(
            x_hbm.at[c].at[edges_v.at[b, 0]], rows_v.at[b], sem_g.at[b]
        ).wait()

    def start_scatter(b):
        pltpu.async_copy(
            rows_v.at[b], acc.at[edges_v.at[b, 1]], sem_s.at[b], add=True
        )

    def wait_scatter(b):
        pltpu.make_async_copy(
            rows_v.at[b], acc.at[edges_v.at[b, 1]], sem_s.at[b]
        ).wait()

    for g in range(LOOK):
        start_gather(g, g % NBUF)

    @pl.loop(0, NCHUNK)
    def _edge_chunk(g):
        b = lax.rem(g, NBUF)
        gl = g + LOOK
        bl = lax.rem(gl, NBUF)

        @pl.when(jnp.logical_and(gl < NCHUNK, gl >= NBUF))
        def _():
            wait_scatter(bl)

        @pl.when(gl < NCHUNK)
        def _():
            start_gather(gl, bl)

        wait_gather(b)

        @pl.loop(0, CHUNK // 16)
        def _scale_group(grp):
            av = edges_v[b, 2, pl.ds(grp * 16, 16)]
            for l in range(16):
                a = lax.bitcast_convert_type(av[l], jnp.float32)
                e = grp * 16 + l
                for j in range(DH // 16):
                    sl = pl.ds(j * 16, 16)
                    rows_v[b, e, sl] = rows_v[b, e, sl] * a

        start_scatter(b)

    for g in range(NCHUNK - NBUF, NCHUNK):
        wait_scatter(g % NBUF)

    plsc.subcore_barrier()

    pltpu.sync_copy(
        acc.at[pl.ds(s * ROWS_PER_TILE, ROWS_PER_TILE)],
        out_hbm.at[c].at[pl.ds(s * ROWS_PER_TILE, ROWS_PER_TILE)],
    )


_sc_agg = pl.kernel(
    _sc_body,
    out_type=jax.ShapeDtypeStruct((NC, NPAD, DH), jnp.float32),
    mesh=plsc.VectorSubcoreMesh(core_axis_name="c", subcore_axis_name="s"),
    compiler_params=pltpu.CompilerParams(use_tc_tiling_on_sc=False),
    scratch_types=[
        pltpu.VMEM((NBUF, 3, CHUNK), jnp.int32),
        pltpu.VMEM((NBUF, CHUNK, DH), jnp.float32),
        pltpu.VMEM_SHARED((NPAD, DH), jnp.float32),
        pltpu.SemaphoreType.DMA((NBUF,)),
        pltpu.SemaphoreType.DMA((NBUF,)),
    ],
)


def _tc_body(part_ref, w_ref, b_ref, gamma_ref, beta_ref, out_ref):
    agg = jnp.concatenate([part_ref[0, 0:N, :], part_ref[1, 0:N, :]], axis=1)
    y = jnp.dot(agg, w_ref[...], preferred_element_type=jnp.float32)
    y = y + b_ref[...]
    mean = jnp.mean(y, axis=0, keepdims=True)
    yc = y - mean
    var = jnp.mean(yc * yc, axis=0, keepdims=True)
    scale = lax.rsqrt(var + 1e-5) * gamma_ref[...]
    out_ref[...] = jnp.maximum(yc * scale + beta_ref[...], 0.0)


@jax.jit
def _run(x, src, dst, attr, W, b, gamma, beta):
    pad = EPAD - E
    packed = jnp.stack(
        [
            jnp.pad(src, (0, pad)),
            jnp.pad(dst, (0, pad)),
            lax.bitcast_convert_type(jnp.pad(attr, (0, pad)), jnp.int32),
        ],
        axis=0,
    )
    packed = packed.reshape(3, NS, NCHUNK, CHUNK).transpose(1, 2, 0, 3)
    x_split = x.reshape(N, NC, DH).transpose(1, 0, 2)

    partial = _sc_agg(packed, x_split)

    out = pl.pallas_call(
        _tc_body,
        out_shape=jax.ShapeDtypeStruct((N, D), jnp.float32),
    )(partial, W, b.reshape(1, D), gamma.reshape(1, D), beta.reshape(1, D))
    return out


def kernel(x, edge_index, edge_attr, batch, W, b, gamma, beta):
    out = _run(x, edge_index[0], edge_index[1], edge_attr, W, b, gamma, beta)
    return (out, edge_index, edge_attr, batch)


# speedup vs baseline: 1.5387x; 1.5387x over previous
import jax
import jax.numpy as jnp
from jax import lax
from jax.experimental import pallas as pl
from jax.experimental.pallas import tpu as pltpu
from jax.experimental.pallas import tpu_sc as plsc

N = 10000
D = 128
E = 320000

NC = 2
NS = 16
NW = NC * NS

CHUNK = 128
EDGES_PER_TILE = -(-E // (NW * CHUNK)) * CHUNK
EPAD = EDGES_PER_TILE * NW
NCHUNK = EDGES_PER_TILE // CHUNK
NBUF = 3
LOOK = 1
ROWS_PER_TILE = -(-N // NS)


def _sc_body(edges_hbm, x_hbm, out_hbm, edges_v, rows_v, acc, sem_g, sem_s):
    c = lax.axis_index("c")
    s = lax.axis_index("s")
    w = c * NS + s

    @pl.loop(0, CHUNK)
    def _zero_rows(i):
        for j in range(D // 16):
            rows_v[0, i, pl.ds(j * 16, 16)] = jnp.zeros((16,), jnp.float32)

    for j in range(ROWS_PER_TILE // CHUNK):
        pltpu.sync_copy(
            rows_v.at[0], acc.at[pl.ds(s * ROWS_PER_TILE + j * CHUNK, CHUNK)]
        )
    rem = ROWS_PER_TILE % CHUNK
    if rem:
        pltpu.sync_copy(
            rows_v.at[0, pl.ds(0, rem)],
            acc.at[pl.ds(s * ROWS_PER_TILE + ROWS_PER_TILE - rem, rem)],
        )
    plsc.subcore_barrier()

    def start_gather(g, b):
        pltpu.sync_copy(edges_hbm.at[w, g], edges_v.at[b])
        pltpu.async_copy(x_hbm.at[edges_v.at[b, 0]], rows_v.at[b], sem_g.at[b])

    def wait_gather(b):
        pltpu.make_async_copy(
            x_hbm.at[edges_v.at[b, 0]], rows_v.at[b], sem_g.at[b]
        ).wait()

    def start_scatter(b):
        pltpu.async_copy(
            rows_v.at[b], acc.at[edges_v.at[b, 1]], sem_s.at[b], add=True
        )

    def wait_scatter(b):
        pltpu.make_async_copy(
            rows_v.at[b], acc.at[edges_v.at[b, 1]], sem_s.at[b]
        ).wait()

    for g in range(LOOK):
        start_gather(g, g % NBUF)

    @pl.loop(0, NCHUNK)
    def _edge_chunk(g):
        b = lax.rem(g, NBUF)
        gl = g + LOOK
        bl = lax.rem(gl, NBUF)

        @pl.when(jnp.logical_and(gl < NCHUNK, gl >= NBUF))
        def _():
            wait_scatter(bl)

        @pl.when(gl < NCHUNK)
        def _():
            start_gather(gl, bl)

        wait_gather(b)

        for grp in range(CHUNK // 16):
            av = edges_v[b, 2, pl.ds(grp * 16, 16)]
            for l in range(16):
                a = lax.bitcast_convert_type(av[l], jnp.float32)
                e = grp * 16 + l
                for j in range(D // 16):
                    sl = pl.ds(j * 16, 16)
                    rows_v[b, e, sl] = rows_v[b, e, sl] * a

        start_scatter(b)

    for g in range(NCHUNK - NBUF, NCHUNK):
        wait_scatter(g % NBUF)

    plsc.subcore_barrier()

    pltpu.sync_copy(
        acc.at[pl.ds(s * ROWS_PER_TILE, ROWS_PER_TILE)],
        out_hbm.at[pl.ds(c * N + s * ROWS_PER_TILE, ROWS_PER_TILE)],
    )


_sc_agg = pl.kernel(
    _sc_body,
    out_type=jax.ShapeDtypeStruct((NC * N, D), jnp.float32),
    mesh=plsc.VectorSubcoreMesh(core_axis_name="c", subcore_axis_name="s"),
    compiler_params=pltpu.CompilerParams(use_tc_tiling_on_sc=False),
    scratch_types=[
        pltpu.VMEM((NBUF, 3, CHUNK), jnp.int32),
        pltpu.VMEM((NBUF, CHUNK, D), jnp.float32),
        pltpu.VMEM_SHARED((N, D), jnp.float32),
        pltpu.SemaphoreType.DMA((NBUF,)),
        pltpu.SemaphoreType.DMA((NBUF,)),
    ],
)


def _tc_body(part_ref, w_ref, b_ref, gamma_ref, beta_ref, out_ref):
    agg = part_ref[0:N, :] + part_ref[N:2 * N, :]
    y = jnp.dot(agg, w_ref[...], preferred_element_type=jnp.float32)
    y = y + b_ref[...]
    mean = jnp.mean(y, axis=0, keepdims=True)
    yc = y - mean
    var = jnp.mean(yc * yc, axis=0, keepdims=True)
    scale = lax.rsqrt(var + 1e-5) * gamma_ref[...]
    out_ref[...] = jnp.maximum(yc * scale + beta_ref[...], 0.0)


@jax.jit
def _run(x, src, dst, attr, W, b, gamma, beta):
    pad = EPAD - E
    packed = jnp.stack(
        [
            jnp.pad(src, (0, pad)),
            jnp.pad(dst, (0, pad)),
            lax.bitcast_convert_type(jnp.pad(attr, (0, pad)), jnp.int32),
        ],
        axis=0,
    )
    packed = packed.reshape(3, NW, NCHUNK, CHUNK).transpose(1, 2, 0, 3)

    partial = _sc_agg(packed, x)

    out = pl.pallas_call(
        _tc_body,
        out_shape=jax.ShapeDtypeStruct((N, D), jnp.float32),
    )(partial, W, b.reshape(1, D), gamma.reshape(1, D), beta.reshape(1, D))
    return out


def kernel(x, edge_index, edge_attr, batch, W, b, gamma, beta):
    out = _run(x, edge_index[0], edge_index[1], edge_attr, W, b, gamma, beta)
    return (out, edge_index, edge_attr, batch)


# speedup vs baseline: 2.8940x; 1.8808x over previous
import jax
import jax.numpy as jnp
from jax import lax
from jax.experimental import pallas as pl
from jax.experimental.pallas import tpu as pltpu
from jax.experimental.pallas import tpu_sc as plsc

N = 10000
D = 128
E = 320000

NC = 2
NS = 16
NW = NC * NS

CHUNK = 128
EDGES_PER_TILE = -(-E // (NW * CHUNK)) * CHUNK
EPAD = EDGES_PER_TILE * NW
NCHUNK = EDGES_PER_TILE // CHUNK
NBUF = 3
LOOK = 1
ROWS_PER_TILE = -(-N // NS)


def _sc_body(edges_hbm, x_hbm, out_hbm, edges_v, rows_v, acc, sem_g, sem_s):
    c = lax.axis_index("c")
    s = lax.axis_index("s")
    w = c * NS + s

    @pl.loop(0, CHUNK)
    def _zero_rows(i):
        for j in range(D // 16):
            rows_v[0, i, pl.ds(j * 16, 16)] = jnp.zeros((16,), jnp.float32)

    for j in range(ROWS_PER_TILE // CHUNK):
        pltpu.sync_copy(
            rows_v.at[0], acc.at[pl.ds(s * ROWS_PER_TILE + j * CHUNK, CHUNK)]
        )
    rem = ROWS_PER_TILE % CHUNK
    if rem:
        pltpu.sync_copy(
            rows_v.at[0, pl.ds(0, rem)],
            acc.at[pl.ds(s * ROWS_PER_TILE + ROWS_PER_TILE - rem, rem)],
        )
    plsc.subcore_barrier()

    def start_gather(g, b):
        pltpu.sync_copy(edges_hbm.at[w, g], edges_v.at[b])
        pltpu.async_copy(x_hbm.at[edges_v.at[b, 0]], rows_v.at[b], sem_g.at[b])

    def wait_gather(b):
        pltpu.make_async_copy(
            x_hbm.at[edges_v.at[b, 0]], rows_v.at[b], sem_g.at[b]
        ).wait()

    def start_scatter(b):
        pltpu.async_copy(
            rows_v.at[b], acc.at[edges_v.at[b, 1]], sem_s.at[b], add=True
        )

    def wait_scatter(b):
        pltpu.make_async_copy(
            rows_v.at[b], acc.at[edges_v.at[b, 1]], sem_s.at[b]
        ).wait()

    for g in range(LOOK):
        start_gather(g, g % NBUF)

    @pl.loop(0, NCHUNK)
    def _edge_chunk(g):
        b = lax.rem(g, NBUF)
        gl = g + LOOK
        bl = lax.rem(gl, NBUF)

        @pl.when(jnp.logical_and(gl < NCHUNK, gl >= NBUF))
        def _():
            wait_scatter(bl)

        @pl.when(gl < NCHUNK)
        def _():
            start_gather(gl, bl)

        wait_gather(b)

        for grp in range(CHUNK // 16):
            av = edges_v[b, 2, pl.ds(grp * 16, 16)]
            for l in range(16):
                a = lax.bitcast_convert_type(av[l], jnp.float32)
                e = grp * 16 + l
                for j in range(D // 16):
                    sl = pl.ds(j * 16, 16)
                    rows_v[b, e, sl] = rows_v[b, e, sl] * a

        start_scatter(b)

    for g in range(NCHUNK - NBUF, NCHUNK):
        wait_scatter(g % NBUF)

    plsc.subcore_barrier()

    pltpu.sync_copy(
        acc.at[pl.ds(s * ROWS_PER_TILE, ROWS_PER_TILE)],
        out_hbm.at[pl.ds(c * N + s * ROWS_PER_TILE, ROWS_PER_TILE)],
    )


_sc_agg = pl.kernel(
    _sc_body,
    out_type=jax.ShapeDtypeStruct((NC * N, D), jnp.float32),
    mesh=plsc.VectorSubcoreMesh(core_axis_name="c", subcore_axis_name="s"),
    compiler_params=pltpu.CompilerParams(use_tc_tiling_on_sc=False),
    scratch_types=[
        pltpu.VMEM((NBUF, 3, CHUNK), jnp.int32),
        pltpu.VMEM((NBUF, CHUNK, D), jnp.float32),
        pltpu.VMEM_SHARED((N, D), jnp.float32),
        pltpu.SemaphoreType.DMA((NBUF,)),
        pltpu.SemaphoreType.DMA((NBUF,)),
    ],
)


def _tc_body(part_ref, w_ref, b_ref, gamma_ref, beta_ref, out_ref):
    agg = part_ref[0:N, :] + part_ref[N:2 * N, :]
    y = jnp.dot(agg, w_ref[...], preferred_element_type=jnp.float32)
    y = y + b_ref[...]
    mean = jnp.mean(y, axis=0, keepdims=True)
    yc = y - mean
    var = jnp.mean(yc * yc, axis=0, keepdims=True)
    scale = lax.rsqrt(var + 1e-5) * gamma_ref[...]
    out_ref[...] = jnp.maximum(yc * scale + beta_ref[...], 0.0)


@jax.jit
def _run(x, src, dst, attr, W, b, gamma, beta):
    pad = EPAD - E
    spread = jnp.arange(pad, dtype=jnp.int32) % N
    packed = jnp.stack(
        [
            jnp.concatenate([src, spread]),
            jnp.concatenate([dst, spread]),
            lax.bitcast_convert_type(jnp.pad(attr, (0, pad)), jnp.int32),
        ],
        axis=0,
    )
    packed = packed.reshape(3, NW, NCHUNK, CHUNK).transpose(1, 2, 0, 3)

    partial = _sc_agg(packed, x)

    out = pl.pallas_call(
        _tc_body,
        out_shape=jax.ShapeDtypeStruct((N, D), jnp.float32),
    )(partial, W, b.reshape(1, D), gamma.reshape(1, D), beta.reshape(1, D))
    return out


def kernel(x, edge_index, edge_attr, batch, W, b, gamma, beta):
    out = _run(x, edge_index[0], edge_index[1], edge_attr, W, b, gamma, beta)
    return (out, edge_index, edge_attr, batch)


# speedup vs baseline: 2.9235x; 1.0102x over previous
import jax
import jax.numpy as jnp
from jax import lax
from jax.experimental import pallas as pl
from jax.experimental.pallas import tpu as pltpu
from jax.experimental.pallas import tpu_sc as plsc

N = 10000
D = 128
E = 320000

NC = 2
NS = 16
NW = NC * NS

CHUNK = 128
EDGES_PER_TILE = -(-E // (NW * CHUNK)) * CHUNK
EPAD = EDGES_PER_TILE * NW
NCHUNK = EDGES_PER_TILE // CHUNK
NBUF = 3
NBI = 4
ROWS_PER_TILE = -(-N // NS)

BLK = 1000
NB = N // BLK


def _sc_body(edges_hbm, x_hbm, out_hbm, edges_v, rows_v, acc, sem_i, sem_g, sem_s):
    c = lax.axis_index("c")
    s = lax.axis_index("s")
    w = c * NS + s

    @pl.loop(0, CHUNK)
    def _zero_rows(i):
        for j in range(D // 16):
            rows_v[0, i, pl.ds(j * 16, 16)] = jnp.zeros((16,), jnp.float32)

    for j in range(ROWS_PER_TILE // CHUNK):
        pltpu.sync_copy(
            rows_v.at[0], acc.at[pl.ds(s * ROWS_PER_TILE + j * CHUNK, CHUNK)]
        )
    rem = ROWS_PER_TILE % CHUNK
    if rem:
        pltpu.sync_copy(
            rows_v.at[0, pl.ds(0, rem)],
            acc.at[pl.ds(s * ROWS_PER_TILE + ROWS_PER_TILE - rem, rem)],
        )
    plsc.subcore_barrier()

    def start_idx(g, bi):
        pltpu.async_copy(edges_hbm.at[w, g], edges_v.at[bi], sem_i.at[bi])

    def wait_idx(g, bi):
        pltpu.make_async_copy(
            edges_hbm.at[w, g], edges_v.at[bi], sem_i.at[bi]
        ).wait()

    def start_gather(b, bi):
        pltpu.async_copy(x_hbm.at[edges_v.at[bi, 0]], rows_v.at[b], sem_g.at[b])

    def wait_gather(b, bi):
        pltpu.make_async_copy(
            x_hbm.at[edges_v.at[bi, 0]], rows_v.at[b], sem_g.at[b]
        ).wait()

    def start_scatter(b, bi):
        pltpu.async_copy(
            rows_v.at[b], acc.at[edges_v.at[bi, 1]], sem_s.at[b], add=True
        )

    def wait_scatter(b, bi):
        pltpu.make_async_copy(
            rows_v.at[b], acc.at[edges_v.at[bi, 1]], sem_s.at[b]
        ).wait()

    start_idx(0, 0)
    start_idx(1, 1)
    wait_idx(0, 0)
    start_gather(0, 0)

    @pl.loop(0, NCHUNK)
    def _edge_chunk(g):
        b = lax.rem(g, NBUF)
        bi = lax.rem(g, NBI)
        b1 = lax.rem(g + 1, NBUF)
        bi1 = lax.rem(g + 1, NBI)
        bi2 = lax.rem(g + 2, NBI)

        @pl.when(g >= 2)
        def _():
            wait_scatter(b1, bi2)

        @pl.when(g + 2 < NCHUNK)
        def _():
            start_idx(g + 2, bi2)

        @pl.when(g + 1 < NCHUNK)
        def _():
            wait_idx(g + 1, bi1)
            start_gather(b1, bi1)

        wait_gather(b, bi)

        for grp in range(CHUNK // 16):
            av = edges_v[bi, 2, pl.ds(grp * 16, 16)]
            for l in range(16):
                a = lax.bitcast_convert_type(av[l], jnp.float32)
                e = grp * 16 + l
                for j in range(D // 16):
                    sl = pl.ds(j * 16, 16)
                    rows_v[b, e, sl] = rows_v[b, e, sl] * a

        start_scatter(b, bi)

    for g in range(NCHUNK - 2, NCHUNK):
        wait_scatter(g % NBUF, g % NBI)

    plsc.subcore_barrier()

    pltpu.sync_copy(
        acc.at[pl.ds(s * ROWS_PER_TILE, ROWS_PER_TILE)],
        out_hbm.at[c].at[pl.ds(s * ROWS_PER_TILE, ROWS_PER_TILE)],
    )


_sc_agg = pl.kernel(
    _sc_body,
    out_type=jax.ShapeDtypeStruct((NC, N, D), jnp.float32),
    mesh=plsc.VectorSubcoreMesh(core_axis_name="c", subcore_axis_name="s"),
    compiler_params=pltpu.CompilerParams(use_tc_tiling_on_sc=False),
    scratch_types=[
        pltpu.VMEM((NBI, 3, CHUNK), jnp.int32),
        pltpu.VMEM((NBUF, CHUNK, D), jnp.float32),
        pltpu.VMEM_SHARED((N, D), jnp.float32),
        pltpu.SemaphoreType.DMA((NBI,)),
        pltpu.SemaphoreType.DMA((NBUF,)),
        pltpu.SemaphoreType.DMA((NBUF,)),
    ],
)


def _tc_a_body(part_ref, w_ref, b_ref, y_ref, stats_ref):
    agg = part_ref[0] + part_ref[1]
    y = jnp.dot(agg, w_ref[...], preferred_element_type=jnp.float32)
    y = y + b_ref[...]
    y_ref[...] = y
    stats_ref[0, 0:1, :] = jnp.sum(y, axis=0, keepdims=True)
    stats_ref[0, 1:2, :] = jnp.sum(y * y, axis=0, keepdims=True)


def _tc_b_body(y_ref, stats_ref, gamma_ref, beta_ref, out_ref):
    stats = stats_ref[...]
    mean = jnp.sum(stats[:, 0, :], axis=0, keepdims=True) * (1.0 / N)
    ex2 = jnp.sum(stats[:, 1, :], axis=0, keepdims=True) * (1.0 / N)
    var = ex2 - mean * mean
    scale = lax.rsqrt(var + 1e-5) * gamma_ref[...]
    shift = beta_ref[...] - mean * scale
    out_ref[...] = jnp.maximum(y_ref[...] * scale + shift, 0.0)


@jax.jit
def _run(x, src, dst, attr, W, b, gamma, beta):
    pad = EPAD - E
    spread = jnp.arange(pad, dtype=jnp.int32) % N
    packed = jnp.stack(
        [
            jnp.concatenate([src, spread]),
            jnp.concatenate([dst, spread]),
            lax.bitcast_convert_type(jnp.pad(attr, (0, pad)), jnp.int32),
        ],
        axis=0,
    )
    packed = packed.reshape(3, NW, NCHUNK, CHUNK).transpose(1, 2, 0, 3)

    partial = _sc_agg(packed, x)

    y, stats = pl.pallas_call(
        _tc_a_body,
        grid=(NB,),
        in_specs=[
            pl.BlockSpec((NC, BLK, D), lambda i: (0, i, 0)),
            pl.BlockSpec((D, D), lambda i: (0, 0)),
            pl.BlockSpec((1, D), lambda i: (0, 0)),
        ],
        out_specs=[
            pl.BlockSpec((BLK, D), lambda i: (i, 0)),
            pl.BlockSpec((1, 2, D), lambda i: (i, 0, 0)),
        ],
        out_shape=[
            jax.ShapeDtypeStruct((N, D), jnp.float32),
            jax.ShapeDtypeStruct((NB, 2, D), jnp.float32),
        ],
    )(partial, W, b.reshape(1, D))

    out = pl.pallas_call(
        _tc_b_body,
        grid=(NB,),
        in_specs=[
            pl.BlockSpec((BLK, D), lambda i: (i, 0)),
            pl.BlockSpec((NB, 2, D), lambda i: (0, 0, 0)),
            pl.BlockSpec((1, D), lambda i: (0, 0)),
            pl.BlockSpec((1, D), lambda i: (0, 0)),
        ],
        out_specs=pl.BlockSpec((BLK, D), lambda i: (i, 0)),
        out_shape=jax.ShapeDtypeStruct((N, D), jnp.float32),
    )(y, stats, gamma.reshape(1, D), beta.reshape(1, D))
    return out


def kernel(x, edge_index, edge_attr, batch, W, b, gamma, beta):
    out = _run(x, edge_index[0], edge_index[1], edge_attr, W, b, gamma, beta)
    return (out, edge_index, edge_attr, batch)


# speedup vs baseline: 3.0291x; 1.0361x over previous
import jax
import jax.numpy as jnp
from jax import lax
from jax.experimental import pallas as pl
from jax.experimental.pallas import tpu as pltpu
from jax.experimental.pallas import tpu_sc as plsc

N = 10000
D = 128
E = 320000

NC = 2
NS = 16
NW = NC * NS

CHUNK = 128
EDGES_PER_TILE = -(-E // (NW * CHUNK)) * CHUNK
EPAD = EDGES_PER_TILE * NW
NCHUNK = EDGES_PER_TILE // CHUNK
NBUF = 3
NBI = 4
ROWS_PER_TILE = -(-N // NS)

BLK = 1000
NB = N // BLK


def _sc_body(src_hbm, dst_hbm, attr_hbm, x_hbm, out_hbm,
             src_v, dst_v, attr_v, rows_v, acc, sem_i, sem_g, sem_s):
    c = lax.axis_index("c")
    s = lax.axis_index("s")
    w = c * NS + s

    @pl.loop(0, CHUNK)
    def _zero_rows(i):
        for j in range(D // 16):
            rows_v[0, i, pl.ds(j * 16, 16)] = jnp.zeros((16,), jnp.float32)

    for j in range(ROWS_PER_TILE // CHUNK):
        pltpu.sync_copy(
            rows_v.at[0], acc.at[pl.ds(s * ROWS_PER_TILE + j * CHUNK, CHUNK)]
        )
    rem = ROWS_PER_TILE % CHUNK
    if rem:
        pltpu.sync_copy(
            rows_v.at[0, pl.ds(0, rem)],
            acc.at[pl.ds(s * ROWS_PER_TILE + ROWS_PER_TILE - rem, rem)],
        )
    plsc.subcore_barrier()

    def start_idx(g, bi):
        pltpu.async_copy(src_hbm.at[w, g], src_v.at[bi], sem_i.at[bi])
        pltpu.async_copy(dst_hbm.at[w, g], dst_v.at[bi], sem_i.at[bi])
        pltpu.async_copy(attr_hbm.at[w, g], attr_v.at[bi], sem_i.at[bi])

    def wait_idx(g, bi):
        pltpu.make_async_copy(src_hbm.at[w, g], src_v.at[bi], sem_i.at[bi]).wait()
        pltpu.make_async_copy(dst_hbm.at[w, g], dst_v.at[bi], sem_i.at[bi]).wait()
        pltpu.make_async_copy(attr_hbm.at[w, g], attr_v.at[bi], sem_i.at[bi]).wait()

    def start_gather(b, bi):
        pltpu.async_copy(x_hbm.at[src_v.at[bi]], rows_v.at[b], sem_g.at[b])

    def wait_gather(b, bi):
        pltpu.make_async_copy(
            x_hbm.at[src_v.at[bi]], rows_v.at[b], sem_g.at[b]
        ).wait()

    def start_scatter(b, bi):
        pltpu.async_copy(
            rows_v.at[b], acc.at[dst_v.at[bi]], sem_s.at[b], add=True
        )

    def wait_scatter(b, bi):
        pltpu.make_async_copy(
            rows_v.at[b], acc.at[dst_v.at[bi]], sem_s.at[b]
        ).wait()

    start_idx(0, 0)
    start_idx(1, 1)
    wait_idx(0, 0)
    start_gather(0, 0)

    @pl.loop(0, NCHUNK)
    def _edge_chunk(g):
        b = lax.rem(g, NBUF)
        bi = lax.rem(g, NBI)
        b1 = lax.rem(g + 1, NBUF)
        bi1 = lax.rem(g + 1, NBI)
        bi2 = lax.rem(g + 2, NBI)

        @pl.when(g >= 2)
        def _():
            wait_scatter(b1, bi2)

        @pl.when(g + 2 < NCHUNK)
        def _():
            start_idx(g + 2, bi2)

        @pl.when(g + 1 < NCHUNK)
        def _():
            wait_idx(g + 1, bi1)
            start_gather(b1, bi1)

        wait_gather(b, bi)

        for grp in range(CHUNK // 16):
            av = attr_v[bi, pl.ds(grp * 16, 16)]
            for l in range(16):
                a = av[l]
                e = grp * 16 + l
                for j in range(D // 16):
                    sl = pl.ds(j * 16, 16)
                    rows_v[b, e, sl] = rows_v[b, e, sl] * a

        start_scatter(b, bi)

    for g in range(NCHUNK - 2, NCHUNK):
        wait_scatter(g % NBUF, g % NBI)

    plsc.subcore_barrier()

    pltpu.sync_copy(
        acc.at[pl.ds(s * ROWS_PER_TILE, ROWS_PER_TILE)],
        out_hbm.at[c].at[pl.ds(s * ROWS_PER_TILE, ROWS_PER_TILE)],
    )


_sc_agg = pl.kernel(
    _sc_body,
    out_type=jax.ShapeDtypeStruct((NC, N, D), jnp.float32),
    mesh=plsc.VectorSubcoreMesh(core_axis_name="c", subcore_axis_name="s"),
    compiler_params=pltpu.CompilerParams(use_tc_tiling_on_sc=False),
    scratch_types=[
        pltpu.VMEM((NBI, CHUNK), jnp.int32),
        pltpu.VMEM((NBI, CHUNK), jnp.int32),
        pltpu.VMEM((NBI, CHUNK), jnp.float32),
        pltpu.VMEM((NBUF, CHUNK, D), jnp.float32),
        pltpu.VMEM_SHARED((N, D), jnp.float32),
        pltpu.SemaphoreType.DMA((NBI,)),
        pltpu.SemaphoreType.DMA((NBUF,)),
        pltpu.SemaphoreType.DMA((NBUF,)),
    ],
)


def _tc_a_body(part_ref, w_ref, b_ref, y_ref, stats_ref):
    agg = part_ref[0] + part_ref[1]
    y = jnp.dot(agg, w_ref[...], preferred_element_type=jnp.float32)
    y = y + b_ref[...]
    y_ref[...] = y
    stats_ref[0, 0:1, :] = jnp.sum(y, axis=0, keepdims=True)
    stats_ref[0, 1:2, :] = jnp.sum(y * y, axis=0, keepdims=True)


def _tc_b_body(y_ref, stats_ref, gamma_ref, beta_ref, out_ref):
    stats = stats_ref[...]
    mean = jnp.sum(stats[:, 0, :], axis=0, keepdims=True) * (1.0 / N)
    ex2 = jnp.sum(stats[:, 1, :], axis=0, keepdims=True) * (1.0 / N)
    var = ex2 - mean * mean
    scale = lax.rsqrt(var + 1e-5) * gamma_ref[...]
    shift = beta_ref[...] - mean * scale
    out_ref[...] = jnp.maximum(y_ref[...] * scale + shift, 0.0)


@jax.jit
def _run(x, src, dst, attr, W, b, gamma, beta):
    pad = EPAD - E
    spread = jnp.arange(pad, dtype=jnp.int32) % N
    src_p = jnp.concatenate([src, spread]).reshape(NW, NCHUNK, CHUNK)
    dst_p = jnp.concatenate([dst, spread]).reshape(NW, NCHUNK, CHUNK)
    attr_p = jnp.pad(attr, (0, pad)).reshape(NW, NCHUNK, CHUNK)

    partial = _sc_agg(src_p, dst_p, attr_p, x)

    y, stats = pl.pallas_call(
        _tc_a_body,
        grid=(NB,),
        in_specs=[
            pl.BlockSpec((NC, BLK, D), lambda i: (0, i, 0)),
            pl.BlockSpec((D, D), lambda i: (0, 0)),
            pl.BlockSpec((1, D), lambda i: (0, 0)),
        ],
        out_specs=[
            pl.BlockSpec((BLK, D), lambda i: (i, 0)),
            pl.BlockSpec((1, 2, D), lambda i: (i, 0, 0)),
        ],
        out_shape=[
            jax.ShapeDtypeStruct((N, D), jnp.float32),
            jax.ShapeDtypeStruct((NB, 2, D), jnp.float32),
        ],
    )(partial, W, b.reshape(1, D))

    out = pl.pallas_call(
        _tc_b_body,
        grid=(NB,),
        in_specs=[
            pl.BlockSpec((BLK, D), lambda i: (i, 0)),
            pl.BlockSpec((NB, 2, D), lambda i: (0, 0, 0)),
            pl.BlockSpec((1, D), lambda i: (0, 0)),
            pl.BlockSpec((1, D), lambda i: (0, 0)),
        ],
        out_specs=pl.BlockSpec((BLK, D), lambda i: (i, 0)),
        out_shape=jax.ShapeDtypeStruct((N, D), jnp.float32),
    )(y, stats, gamma.reshape(1, D), beta.reshape(1, D))
    return out


def kernel(x, edge_index, edge_attr, batch, W, b, gamma, beta):
    out = _run(x, edge_index[0], edge_index[1], edge_attr, W, b, gamma, beta)
    return (out, edge_index, edge_attr, batch)


# speedup vs baseline: 3.0564x; 1.0090x over previous
import jax
import jax.numpy as jnp
from jax import lax
from jax.experimental import pallas as pl
from jax.experimental.pallas import tpu as pltpu
from jax.experimental.pallas import tpu_sc as plsc

N = 10000
D = 128
E = 320000

NC = 2
NS = 16
NW = NC * NS

CHUNK = 128
EDGES_PER_TILE = -(-E // (NW * CHUNK)) * CHUNK
NCHUNK = EDGES_PER_TILE // CHUNK
LAST_NCHUNK = (E - (NW - 1) * EDGES_PER_TILE) // CHUNK
assert (E - (NW - 1) * EDGES_PER_TILE) % CHUNK == 0 and LAST_NCHUNK >= 2
NBUF = 3
NBI = 4
ROWS_PER_TILE = -(-N // NS)

BLK = 1000
NB = N // BLK


def _sc_body(src_hbm, dst_hbm, attr_hbm, x_hbm, out_hbm,
             src_v, dst_v, attr_v, rows_v, acc, sem_i, sem_g, sem_s):
    c = lax.axis_index("c")
    s = lax.axis_index("s")
    w = c * NS + s
    nchunk = jnp.where(w == NW - 1, LAST_NCHUNK, NCHUNK)
    ebase = w * EDGES_PER_TILE

    def start_idx(g, bi):
        base = ebase + g * CHUNK
        pltpu.async_copy(src_hbm.at[pl.ds(base, CHUNK)], src_v.at[bi], sem_i.at[bi])
        pltpu.async_copy(dst_hbm.at[pl.ds(base, CHUNK)], dst_v.at[bi], sem_i.at[bi])
        pltpu.async_copy(attr_hbm.at[pl.ds(base, CHUNK)], attr_v.at[bi], sem_i.at[bi])

    def wait_idx(g, bi):
        base = ebase + g * CHUNK
        pltpu.make_async_copy(src_hbm.at[pl.ds(base, CHUNK)], src_v.at[bi], sem_i.at[bi]).wait()
        pltpu.make_async_copy(dst_hbm.at[pl.ds(base, CHUNK)], dst_v.at[bi], sem_i.at[bi]).wait()
        pltpu.make_async_copy(attr_hbm.at[pl.ds(base, CHUNK)], attr_v.at[bi], sem_i.at[bi]).wait()

    start_idx(0, 0)
    start_idx(1, 1)

    @pl.loop(0, CHUNK)
    def _zero_rows(i):
        for j in range(D // 16):
            rows_v[0, i, pl.ds(j * 16, 16)] = jnp.zeros((16,), jnp.float32)

    for j in range(ROWS_PER_TILE // CHUNK):
        pltpu.sync_copy(
            rows_v.at[0], acc.at[pl.ds(s * ROWS_PER_TILE + j * CHUNK, CHUNK)]
        )
    rem = ROWS_PER_TILE % CHUNK
    if rem:
        pltpu.sync_copy(
            rows_v.at[0, pl.ds(0, rem)],
            acc.at[pl.ds(s * ROWS_PER_TILE + ROWS_PER_TILE - rem, rem)],
        )
    plsc.subcore_barrier()

    def start_gather(b, bi):
        pltpu.async_copy(x_hbm.at[src_v.at[bi]], rows_v.at[b], sem_g.at[b])

    def wait_gather(b, bi):
        pltpu.make_async_copy(
            x_hbm.at[src_v.at[bi]], rows_v.at[b], sem_g.at[b]
        ).wait()

    def start_scatter(b, bi):
        pltpu.async_copy(
            rows_v.at[b], acc.at[dst_v.at[bi]], sem_s.at[b], add=True
        )

    def wait_scatter(b, bi):
        pltpu.make_async_copy(
            rows_v.at[b], acc.at[dst_v.at[bi]], sem_s.at[b]
        ).wait()

    wait_idx(0, 0)
    start_gather(0, 0)

    @pl.loop(0, NCHUNK)
    def _edge_chunk(g):
        b = lax.rem(g, NBUF)
        bi = lax.rem(g, NBI)
        b1 = lax.rem(g + 1, NBUF)
        bi1 = lax.rem(g + 1, NBI)
        bi2 = lax.rem(g + 2, NBI)

        @pl.when(jnp.logical_and(g >= 2, g < nchunk))
        def _():
            wait_scatter(b1, bi2)

        @pl.when(g + 2 < nchunk)
        def _():
            start_idx(g + 2, bi2)

        @pl.when(g + 1 < nchunk)
        def _():
            wait_idx(g + 1, bi1)
            start_gather(b1, bi1)

        @pl.when(g < nchunk)
        def _():
            wait_gather(b, bi)

            for grp in range(CHUNK // 16):
                av = attr_v[bi, pl.ds(grp * 16, 16)]
                for l in range(16):
                    a = av[l]
                    e = grp * 16 + l
                    for j in range(D // 16):
                        sl = pl.ds(j * 16, 16)
                        rows_v[b, e, sl] = rows_v[b, e, sl] * a

            start_scatter(b, bi)

    for k in range(2, 0, -1):
        g = nchunk - k
        wait_scatter(lax.rem(g, NBUF), lax.rem(g, NBI))

    plsc.subcore_barrier()

    pltpu.sync_copy(
        acc.at[pl.ds(s * ROWS_PER_TILE, ROWS_PER_TILE)],
        out_hbm.at[c].at[pl.ds(s * ROWS_PER_TILE, ROWS_PER_TILE)],
    )


_sc_agg = pl.kernel(
    _sc_body,
    out_type=jax.ShapeDtypeStruct((NC, N, D), jnp.float32),
    mesh=plsc.VectorSubcoreMesh(core_axis_name="c", subcore_axis_name="s"),
    compiler_params=pltpu.CompilerParams(use_tc_tiling_on_sc=False),
    scratch_types=[
        pltpu.VMEM((NBI, CHUNK), jnp.int32),
        pltpu.VMEM((NBI, CHUNK), jnp.int32),
        pltpu.VMEM((NBI, CHUNK), jnp.float32),
        pltpu.VMEM((NBUF, CHUNK, D), jnp.float32),
        pltpu.VMEM_SHARED((N, D), jnp.float32),
        pltpu.SemaphoreType.DMA((NBI,)),
        pltpu.SemaphoreType.DMA((NBUF,)),
        pltpu.SemaphoreType.DMA((NBUF,)),
    ],
)


def _tc_a_body(part_ref, w_ref, b_ref, y_ref, stats_ref):
    agg = part_ref[0] + part_ref[1]
    y = jnp.dot(agg, w_ref[...], preferred_element_type=jnp.float32)
    y = y + b_ref[...]
    y_ref[...] = y
    stats_ref[0, 0:1, :] = jnp.sum(y, axis=0, keepdims=True)
    stats_ref[0, 1:2, :] = jnp.sum(y * y, axis=0, keepdims=True)


def _tc_b_body(y_ref, stats_ref, gamma_ref, beta_ref, out_ref):
    stats = stats_ref[...]
    mean = jnp.sum(stats[:, 0, :], axis=0, keepdims=True) * (1.0 / N)
    ex2 = jnp.sum(stats[:, 1, :], axis=0, keepdims=True) * (1.0 / N)
    var = ex2 - mean * mean
    scale = lax.rsqrt(var + 1e-5) * gamma_ref[...]
    shift = beta_ref[...] - mean * scale
    out_ref[...] = jnp.maximum(y_ref[...] * scale + shift, 0.0)


@jax.jit
def _run(x, src, dst, attr, W, b, gamma, beta):
    partial = _sc_agg(src, dst, attr, x)

    y, stats = pl.pallas_call(
        _tc_a_body,
        grid=(NB,),
        in_specs=[
            pl.BlockSpec((NC, BLK, D), lambda i: (0, i, 0)),
            pl.BlockSpec((D, D), lambda i: (0, 0)),
            pl.BlockSpec((1, D), lambda i: (0, 0)),
        ],
        out_specs=[
            pl.BlockSpec((BLK, D), lambda i: (i, 0)),
            pl.BlockSpec((1, 2, D), lambda i: (i, 0, 0)),
        ],
        out_shape=[
            jax.ShapeDtypeStruct((N, D), jnp.float32),
            jax.ShapeDtypeStruct((NB, 2, D), jnp.float32),
        ],
    )(partial, W, b.reshape(1, D))

    out = pl.pallas_call(
        _tc_b_body,
        grid=(NB,),
        in_specs=[
            pl.BlockSpec((BLK, D), lambda i: (i, 0)),
            pl.BlockSpec((NB, 2, D), lambda i: (0, 0, 0)),
            pl.BlockSpec((1, D), lambda i: (0, 0)),
            pl.BlockSpec((1, D), lambda i: (0, 0)),
        ],
        out_specs=pl.BlockSpec((BLK, D), lambda i: (i, 0)),
        out_shape=jax.ShapeDtypeStruct((N, D), jnp.float32),
    )(y, stats, gamma.reshape(1, D), beta.reshape(1, D))
    return out


def kernel(x, edge_index, edge_attr, batch, W, b, gamma, beta):
    out = _run(x, edge_index[0], edge_index[1], edge_attr, W, b, gamma, beta)
    return (out, edge_index, edge_attr, batch)
